# baseline (speedup 1.0000x reference)
import functools

import jax
import jax.numpy as jnp
from jax import lax
from jax.experimental import pallas as pl
from jax.experimental.pallas import tpu as pltpu
from jax.experimental.pallas import tpu_sc as plsc

B = 32
N = 4096
K = 32
BN = B * N
NW = 32
CH = 256
NCHUNK = N // CH
NP = NCHUNK // 2
NB = CH // 16
OW = 30

_mesh = plsc.VectorSubcoreMesh(core_axis_name="c", subcore_axis_name="s")


def _rsqrt(v):
    vh = v * 0.5
    i = lax.bitcast_convert_type(v, jnp.int32)
    i = jnp.int32(0x5F3759DF) - lax.shift_right_logical(i, 1)
    y = lax.bitcast_convert_type(i, jnp.float32)
    for _ in range(2):
        y = y * (1.5 - vh * y * y)
    return y


@functools.partial(
    pl.kernel,
    out_type=jax.ShapeDtypeStruct((OW, BN), jnp.float32),
    mesh=_mesh,
    scratch_types=[
        pltpu.VMEM((3, K, CH), jnp.float32),
        pltpu.VMEM((3, K, CH), jnp.float32),
        pltpu.VMEM((OW, CH), jnp.float32),
        pltpu.VMEM((OW, CH), jnp.float32),
        pltpu.VMEM((512,), jnp.float32),
        pltpu.VMEM((3, CH), jnp.float32),
        pltpu.VMEM((3, CH), jnp.float32),
        pltpu.SemaphoreType.DMA,
        pltpu.SemaphoreType.DMA,
        pltpu.SemaphoreType.DMA,
        pltpu.SemaphoreType.DMA,
        pltpu.SemaphoreType.DMA,
    ],
    compiler_params=pltpu.CompilerParams(needs_layout_passes=False),
)
def _pointhop_sc(gx, nc, out, in0, in1, ob0, ob1, acc, sqb, smb,
                 sin0, sin1, scen, sob0, sob1):
    wid = lax.axis_index("s") * 2 + lax.axis_index("c")
    lane = jnp.arange(16, dtype=jnp.int32)
    zeros16 = jnp.zeros((16,), jnp.float32)
    ones16 = jnp.ones((16,), jnp.float32)
    lane256 = lane + 256
    acc_y = acc.at[pl.ds(16, 496)]
    acc_z = acc.at[pl.ds(32, 480)]
    acc_n = acc.at[pl.ds(48, 464)]

    def in_copy(c, ibuf, sem):
        return pltpu.make_async_copy(
            gx.at[wid, :, :, pl.ds(c * CH, CH)], ibuf, sem)

    def batch_body_for(ibuf, obuf):
        def batch_body(b, carry):
            g0 = b * 16
            z3 = (zeros16,) * 3

            @plsc.parallel_loop(0, K, 1, unroll=16, carry=z3)
            def sums_sq(t, csum):
                sxx, syy, szz = csum
                xs = ibuf[0, t, pl.ds(g0, 16)]
                ys = ibuf[1, t, pl.ds(g0, 16)]
                zs = ibuf[2, t, pl.ds(g0, 16)]
                soff = (jnp.where(xs > 0, lane256, lane)
                        + jnp.where(ys > 0, 128, 0)
                        + jnp.where(zs > 0, 64, 0))
                plsc.addupdate_scatter(acc, [soff], xs)
                plsc.addupdate_scatter(acc_y, [soff], ys)
                plsc.addupdate_scatter(acc_z, [soff], zs)
                plsc.addupdate_scatter(acc_n, [soff], ones16)
                return (sxx + xs * xs, syy + ys * ys, szz + zs * zs)

            sxx, syy, szz = sums_sq

            @plsc.parallel_loop(0, 8, 1, unroll=8, carry=z3)
            def totals(o, tot):
                tx, ty, tz = tot
                o64 = o * 64
                cnt = acc[pl.ds(o64 + 48, 16)]
                inv = 1.0 / jnp.maximum(cnt, 1.0)
                bx = acc[pl.ds(o64, 16)]
                by = acc[pl.ds(o64 + 16, 16)]
                bz = acc[pl.ds(o64 + 32, 16)]
                acc[pl.ds(o64, 16)] = zeros16
                acc[pl.ds(o64 + 16, 16)] = zeros16
                acc[pl.ds(o64 + 32, 16)] = zeros16
                acc[pl.ds(o64 + 48, 16)] = zeros16
                o3 = 6 + o * 3
                obuf[o3, pl.ds(g0, 16)] = bx * inv
                obuf[o3 + 1, pl.ds(g0, 16)] = by * inv
                obuf[o3 + 2, pl.ds(g0, 16)] = bz * inv
                return (tx + bx, ty + by, tz + bz)

            for c, s, sq in zip(range(3), totals, (sxx, syy, szz)):
                sqb[c, pl.ds(g0, 16)] = sq
                smb[c, pl.ds(g0, 16)] = s
            return carry
        return batch_body

    body0 = batch_body_for(in0, ob0)
    body1 = batch_body_for(in1, ob1)

    def out_copy(c, obuf, sem):
        return pltpu.make_async_copy(
            obuf, out.at[:, pl.ds(wid * N + c * CH, CH)], sem)

    def do_chunk(c, ibuf, sem, obuf, osem, body, prefetch, first):
        @pl.when(jnp.logical_not(first))
        def _():
            out_copy(c, obuf, osem).wait()
        cen = pltpu.async_copy(nc.at[:, wid, pl.ds(c * CH, CH)],
                               obuf.at[pl.ds(3, 3), :], scen)
        if prefetch is not None:
            prefetch()
        in_copy(c, ibuf, sem).wait()
        lax.fori_loop(0, NB, body, 0, unroll=False)

        @plsc.parallel_loop(0, NB, 1, unroll=4)
        def _(b):
            g0 = b * 16
            for c in range(3):
                sq = sqb[c, pl.ds(g0, 16)]
                s = smb[c, pl.ds(g0, 16)]
                var = sq * (1.0 / (K - 1)) - (s * s) * (1.0 / (K * (K - 1)))
                var = jnp.maximum(var, 0.0)
                obuf[c, pl.ds(g0, 16)] = var * _rsqrt(var)

        cen.wait()
        out_copy(c, obuf, osem).start()

    def pair_body(p, carry):
        c0 = p * 2
        first = p == 0
        do_chunk(c0, in0, sin0, ob0, sob0, body0,
                 lambda: in_copy(c0 + 1, in1, sin1).start(), first)
        @pl.when(p + 1 < NP)
        def _():
            in_copy(c0 + 2, in0, sin0).start()
        do_chunk(c0 + 1, in1, sin1, ob1, sob1, body1, None, first)
        return carry

    for i in range(32):
        acc[pl.ds(i * 16, 16)] = zeros16
    in_copy(0, in0, sin0).start()
    lax.fori_loop(0, NP, pair_body, 0, unroll=False)
    out_copy(NCHUNK - 2, ob0, sob0).wait()
    out_copy(NCHUNK - 1, ob1, sob1).wait()


def kernel(group_xyz, new_xyz):
    gxt = jnp.transpose(group_xyz, (0, 3, 2, 1))
    nct = jnp.transpose(new_xyz, (2, 0, 1))
    out = _pointhop_sc(gxt, nct)
    return out.T


# speedup vs baseline: 1.0669x; 1.0669x over previous
import functools

import jax
import jax.numpy as jnp
from jax import lax
from jax.experimental import pallas as pl
from jax.experimental.pallas import tpu as pltpu
from jax.experimental.pallas import tpu_sc as plsc

B = 32
N = 4096
K = 32
BN = B * N
NW = 32
CH = 256
NCHUNK = N // CH
NP = NCHUNK // 2
NB = CH // 16
OW = 30

_mesh = plsc.VectorSubcoreMesh(core_axis_name="c", subcore_axis_name="s")


def _rsqrt(v):
    vh = v * 0.5
    i = lax.bitcast_convert_type(v, jnp.int32)
    i = jnp.int32(0x5F3759DF) - lax.shift_right_logical(i, 1)
    y = lax.bitcast_convert_type(i, jnp.float32)
    for _ in range(2):
        y = y * (1.5 - vh * y * y)
    return y


@functools.partial(
    pl.kernel,
    out_type=jax.ShapeDtypeStruct((OW, BN), jnp.float32),
    mesh=_mesh,
    scratch_types=[
        pltpu.VMEM((3, K, CH), jnp.float32),
        pltpu.VMEM((3, K, CH), jnp.float32),
        pltpu.VMEM((OW, CH), jnp.float32),
        pltpu.VMEM((OW, CH), jnp.float32),
        pltpu.VMEM((512,), jnp.float32),
        pltpu.SemaphoreType.DMA,
        pltpu.SemaphoreType.DMA,
        pltpu.SemaphoreType.DMA,
        pltpu.SemaphoreType.DMA,
        pltpu.SemaphoreType.DMA,
    ],
    compiler_params=pltpu.CompilerParams(needs_layout_passes=False),
)
def _pointhop_sc(gx, nc, out, in0, in1, ob0, ob1, acc,
                 sin0, sin1, scen, sob0, sob1):
    wid = lax.axis_index("s") * 2 + lax.axis_index("c")
    lane = jnp.arange(16, dtype=jnp.int32)
    zeros16 = jnp.zeros((16,), jnp.float32)
    ones16 = jnp.ones((16,), jnp.float32)
    lane256 = lane + 256
    acc_y = acc.at[pl.ds(16, 496)]
    acc_z = acc.at[pl.ds(32, 480)]
    acc_n = acc.at[pl.ds(48, 464)]

    def in_copy(c, ibuf, sem):
        return pltpu.make_async_copy(
            gx.at[wid, :, :, pl.ds(c * CH, CH)], ibuf, sem)

    def batch_body_for(ibuf, obuf):
        def batch_body(b, carry):
            g0 = b * 16
            z3 = (zeros16,) * 3

            @plsc.parallel_loop(0, K, 1, unroll=16, carry=z3)
            def sums_sq(t, csum):
                sxx, syy, szz = csum
                xs = ibuf[0, t, pl.ds(g0, 16)]
                ys = ibuf[1, t, pl.ds(g0, 16)]
                zs = ibuf[2, t, pl.ds(g0, 16)]
                soff = (jnp.where(xs > 0, lane256, lane)
                        + jnp.where(ys > 0, 128, 0)
                        + jnp.where(zs > 0, 64, 0))
                plsc.addupdate_scatter(acc, [soff], xs)
                plsc.addupdate_scatter(acc_y, [soff], ys)
                plsc.addupdate_scatter(acc_z, [soff], zs)
                plsc.addupdate_scatter(acc_n, [soff], ones16)
                return (sxx + xs * xs, syy + ys * ys, szz + zs * zs)

            sxx, syy, szz = sums_sq

            @plsc.parallel_loop(0, 8, 1, unroll=8, carry=z3)
            def totals(o, tot):
                tx, ty, tz = tot
                o64 = o * 64
                cnt = acc[pl.ds(o64 + 48, 16)]
                inv = 1.0 / jnp.maximum(cnt, 1.0)
                bx = acc[pl.ds(o64, 16)]
                by = acc[pl.ds(o64 + 16, 16)]
                bz = acc[pl.ds(o64 + 32, 16)]
                acc[pl.ds(o64, 16)] = zeros16
                acc[pl.ds(o64 + 16, 16)] = zeros16
                acc[pl.ds(o64 + 32, 16)] = zeros16
                acc[pl.ds(o64 + 48, 16)] = zeros16
                o3 = 6 + o * 3
                obuf[o3, pl.ds(g0, 16)] = bx * inv
                obuf[o3 + 1, pl.ds(g0, 16)] = by * inv
                obuf[o3 + 2, pl.ds(g0, 16)] = bz * inv
                return (tx + bx, ty + by, tz + bz)

            for c, s, sq in zip(range(3), totals, (sxx, syy, szz)):
                var = sq * (1.0 / (K - 1)) - (s * s) * (1.0 / (K * (K - 1)))
                var = jnp.maximum(var, 0.0)
                obuf[c, pl.ds(g0, 16)] = var * _rsqrt(var)
            return carry
        return batch_body

    body0 = batch_body_for(in0, ob0)
    body1 = batch_body_for(in1, ob1)

    def out_copy(c, obuf, sem):
        return pltpu.make_async_copy(
            obuf, out.at[:, pl.ds(wid * N + c * CH, CH)], sem)

    def do_chunk(c, ibuf, sem, obuf, osem, body, prefetch, first):
        @pl.when(jnp.logical_not(first))
        def _():
            out_copy(c, obuf, osem).wait()
        cen = pltpu.async_copy(nc.at[:, wid, pl.ds(c * CH, CH)],
                               obuf.at[pl.ds(3, 3), :], scen)
        if prefetch is not None:
            prefetch()
        in_copy(c, ibuf, sem).wait()
        lax.fori_loop(0, NB, body, 0, unroll=False)
        cen.wait()
        out_copy(c, obuf, osem).start()

    def pair_body(p, carry):
        c0 = p * 2
        first = p == 0
        do_chunk(c0, in0, sin0, ob0, sob0, body0,
                 lambda: in_copy(c0 + 1, in1, sin1).start(), first)
        @pl.when(p + 1 < NP)
        def _():
            in_copy(c0 + 2, in0, sin0).start()
        do_chunk(c0 + 1, in1, sin1, ob1, sob1, body1, None, first)
        return carry

    for i in range(32):
        acc[pl.ds(i * 16, 16)] = zeros16
    in_copy(0, in0, sin0).start()
    lax.fori_loop(0, NP, pair_body, 0, unroll=False)
    out_copy(NCHUNK - 2, ob0, sob0).wait()
    out_copy(NCHUNK - 1, ob1, sob1).wait()


def kernel(group_xyz, new_xyz):
    gxt = jnp.transpose(group_xyz, (0, 3, 2, 1))
    nct = jnp.transpose(new_xyz, (2, 0, 1))
    out = _pointhop_sc(gxt, nct)
    return out.T
